# baseline (speedup 1.0000x reference)
import functools

import jax
import jax.numpy as jnp
from jax import lax
from jax.experimental import pallas as pl
from jax.experimental.pallas import tpu as pltpu
from jax.experimental.pallas import tpu_sc as plsc

N_TOKENS = 32768
DIM = 768
N_EXPERTS = 64
TOPK = 8
N_GROUPS = 8
GROUP_SIZE = N_EXPERTS // N_GROUPS
TOPK_GROUPS = 4

TILE = 4096

NEG_INF = float("-inf")


def _softmax_kernel(x_ref, wt_ref, p_ref):
    scores = jnp.dot(x_ref[...], wt_ref[...], preferred_element_type=jnp.float32)
    s = scores.T
    smax = jnp.max(s, axis=0, keepdims=True)
    e = jnp.exp(s - smax)
    p = e / jnp.sum(e, axis=0, keepdims=True)
    p_ref[...] = p.T


def _route_body(p_hbm, w_hbm, i_hbm, p_v, w_v, i_v, *, rows_per, n_cores):
    cid = lax.axis_index("c")
    sid = lax.axis_index("s")
    wid = sid * n_cores + cid
    base = wid * rows_per

    pltpu.sync_copy(p_hbm.at[pl.ds(base * N_EXPERTS, rows_per * N_EXPERTS)], p_v)

    lane = lax.iota(jnp.int32, 16)
    lane8 = lane & 7
    low8 = lane < 8
    gcol = jnp.where(low8, lane * GROUP_SIZE, 0)

    @plsc.parallel_loop(0, rows_per, unroll=2)
    def _(r):
        rbase = jnp.full((16,), r * N_EXPERTS, jnp.int32)
        gmax = plsc.load_gather(p_v, [rbase + gcol])
        for j in range(1, GROUP_SIZE):
            gmax = jnp.maximum(gmax, plsc.load_gather(p_v, [rbase + gcol + j]))
        gkey = jnp.where(low8, gmax, NEG_INF)
        _, gsel = plsc.sort_key_val(gkey, lane, descending=True)
        ga = gsel.at[lane >> 3].get(mode="promise_in_bounds")
        gb = gsel.at[(lane >> 3) + 2].get(mode="promise_in_bounds")
        cols_a = ga * GROUP_SIZE + lane8
        cols_b = gb * GROUP_SIZE + lane8
        va = plsc.load_gather(p_v, [rbase + cols_a])
        vb = plsc.load_gather(p_v, [rbase + cols_b])
        ka, ia = plsc.sort_key_val(va, cols_a, descending=True)
        kb, ib = plsc.sort_key_val(vb, cols_b, descending=True)
        kc = jnp.where(low8, ka, kb.at[lane8].get(mode="promise_in_bounds"))
        ic = jnp.where(low8, ia, ib.at[lane8].get(mode="promise_in_bounds"))
        kf, if_ = plsc.sort_key_val(kc, ic, descending=True)
        out_idx = jnp.full((16,), r * TOPK, jnp.int32) + lane
        plsc.store_scatter(w_v, [out_idx], kf, mask=low8)
        plsc.store_scatter(i_v, [out_idx], if_, mask=low8)

    pltpu.sync_copy(w_v, w_hbm.at[pl.ds(base * TOPK, rows_per * TOPK)])
    pltpu.sync_copy(i_v, i_hbm.at[pl.ds(base * TOPK, rows_per * TOPK)])


@jax.jit
def kernel(x, weight):
    n = x.shape[0]
    wt = weight.T

    info = plsc.get_sparse_core_info()
    nc, ns = info.num_cores, info.num_subcores
    mesh = plsc.VectorSubcoreMesh(core_axis_name="c", subcore_axis_name="s")

    chunks = 2
    nch = n // chunks
    rows_per = nch // (nc * ns)
    grid = (nch // TILE,)

    ws, is_ = [], []
    for c in range(chunks):
        xc = lax.slice_in_dim(x, c * nch, (c + 1) * nch, axis=0)
        p = pl.pallas_call(
            _softmax_kernel,
            grid=grid,
            in_specs=[
                pl.BlockSpec((TILE, DIM), lambda i: (i, 0)),
                pl.BlockSpec((DIM, N_EXPERTS), lambda i: (0, 0)),
            ],
            out_specs=pl.BlockSpec((TILE, N_EXPERTS), lambda i: (i, 0)),
            out_shape=jax.ShapeDtypeStruct((nch, N_EXPERTS), jnp.float32),
        )(xc, wt)
        w_flat, i_flat = pl.kernel(
            functools.partial(_route_body, rows_per=rows_per, n_cores=nc),
            out_type=[jax.ShapeDtypeStruct((nch * TOPK,), jnp.float32),
                      jax.ShapeDtypeStruct((nch * TOPK,), jnp.int32)],
            mesh=mesh,
            compiler_params=pltpu.CompilerParams(needs_layout_passes=False),
            scratch_types=[pltpu.VMEM((rows_per * N_EXPERTS,), jnp.float32),
                           pltpu.VMEM((rows_per * TOPK,), jnp.float32),
                           pltpu.VMEM((rows_per * TOPK,), jnp.int32)],
        )(p.reshape(-1))
        ws.append(w_flat.reshape(nch, TOPK))
        is_.append(i_flat.reshape(nch, TOPK))
    return (jnp.concatenate(ws, axis=0), jnp.concatenate(is_, axis=0))


# speedup vs baseline: 1.3892x; 1.3892x over previous
import functools

import jax
import jax.numpy as jnp
from jax import lax
from jax.experimental import pallas as pl
from jax.experimental.pallas import tpu as pltpu
from jax.experimental.pallas import tpu_sc as plsc

N_TOKENS = 32768
DIM = 768
N_EXPERTS = 64
TOPK = 8
N_GROUPS = 8
GROUP_SIZE = N_EXPERTS // N_GROUPS
TOPK_GROUPS = 4

TILE = 4096

NEG_INF = float("-inf")


def _softmax_kernel(x_ref, wt_ref, p_ref, g_ref):
    t = x_ref.shape[0]
    scores = jnp.dot(x_ref[...], wt_ref[...], preferred_element_type=jnp.float32)
    s = scores.T
    smax = jnp.max(s, axis=0, keepdims=True)
    e = jnp.exp(s - smax)
    p = e / jnp.sum(e, axis=0, keepdims=True)
    p_ref[...] = p.T
    gs = jnp.max(p.reshape(N_GROUPS, GROUP_SIZE, t), axis=1)
    g_ref[...] = gs.T


def _route_body(p_hbm, g_hbm, w_hbm, i_hbm, p_v, g_v, w_v, i_v, *,
                rows_per, n_cores):
    cid = lax.axis_index("c")
    sid = lax.axis_index("s")
    wid = sid * n_cores + cid
    base = wid * rows_per

    pltpu.sync_copy(p_hbm.at[pl.ds(base * N_EXPERTS, rows_per * N_EXPERTS)], p_v)
    pltpu.sync_copy(g_hbm.at[pl.ds(base * N_GROUPS, rows_per * N_GROUPS)], g_v)

    lane = lax.iota(jnp.int32, 16)
    lane8 = lane & 7
    low8 = lane < 8

    @plsc.parallel_loop(0, rows_per, unroll=2)
    def _(r):
        gmax = plsc.load_gather(g_v, [jnp.full((16,), r * N_GROUPS, jnp.int32)
                                      + lane8])
        gkey = jnp.where(low8, gmax, NEG_INF)
        _, gsel = plsc.sort_key_val(gkey, lane, descending=True)
        ga = gsel.at[lane >> 3].get(mode="promise_in_bounds")
        gb = gsel.at[(lane >> 3) + 2].get(mode="promise_in_bounds")
        cols_a = ga * GROUP_SIZE + lane8
        cols_b = gb * GROUP_SIZE + lane8
        rbase = jnp.full((16,), r * N_EXPERTS, jnp.int32)
        va = plsc.load_gather(p_v, [rbase + cols_a])
        vb = plsc.load_gather(p_v, [rbase + cols_b])
        ka, ia = plsc.sort_key_val(va, cols_a, descending=True)
        kb, ib = plsc.sort_key_val(vb, cols_b, descending=True)
        kc = jnp.where(low8, ka, kb.at[lane8].get(mode="promise_in_bounds"))
        ic = jnp.where(low8, ia, ib.at[lane8].get(mode="promise_in_bounds"))
        kf, if_ = plsc.sort_key_val(kc, ic, descending=True)
        out_idx = jnp.full((16,), r * TOPK, jnp.int32) + lane
        plsc.store_scatter(w_v, [out_idx], kf, mask=low8)
        plsc.store_scatter(i_v, [out_idx], if_, mask=low8)

    pltpu.sync_copy(w_v, w_hbm.at[pl.ds(base * TOPK, rows_per * TOPK)])
    pltpu.sync_copy(i_v, i_hbm.at[pl.ds(base * TOPK, rows_per * TOPK)])


@jax.jit
def kernel(x, weight):
    n = x.shape[0]
    wt = weight.T
    grid = (n // TILE,)
    p, g = pl.pallas_call(
        _softmax_kernel,
        grid=grid,
        in_specs=[
            pl.BlockSpec((TILE, DIM), lambda i: (i, 0)),
            pl.BlockSpec((DIM, N_EXPERTS), lambda i: (0, 0)),
        ],
        out_specs=[
            pl.BlockSpec((TILE, N_EXPERTS), lambda i: (i, 0)),
            pl.BlockSpec((TILE, N_GROUPS), lambda i: (i, 0)),
        ],
        out_shape=[
            jax.ShapeDtypeStruct((n, N_EXPERTS), jnp.float32),
            jax.ShapeDtypeStruct((n, N_GROUPS), jnp.float32),
        ],
    )(x, wt)

    info = plsc.get_sparse_core_info()
    nc, ns = info.num_cores, info.num_subcores
    rows_per = n // (nc * ns)
    mesh = plsc.VectorSubcoreMesh(core_axis_name="c", subcore_axis_name="s")
    w_flat, i_flat = pl.kernel(
        functools.partial(_route_body, rows_per=rows_per, n_cores=nc),
        out_type=[jax.ShapeDtypeStruct((n * TOPK,), jnp.float32),
                  jax.ShapeDtypeStruct((n * TOPK,), jnp.int32)],
        mesh=mesh,
        compiler_params=pltpu.CompilerParams(needs_layout_passes=False),
        scratch_types=[pltpu.VMEM((rows_per * N_EXPERTS,), jnp.float32),
                       pltpu.VMEM((rows_per * N_GROUPS,), jnp.float32),
                       pltpu.VMEM((rows_per * TOPK,), jnp.float32),
                       pltpu.VMEM((rows_per * TOPK,), jnp.int32)],
    )(p.reshape(-1), g.reshape(-1))
    return w_flat.reshape(n, TOPK), i_flat.reshape(n, TOPK)


# speedup vs baseline: 1.3933x; 1.0030x over previous
import functools

import jax
import jax.numpy as jnp
from jax import lax
from jax.experimental import pallas as pl
from jax.experimental.pallas import tpu as pltpu
from jax.experimental.pallas import tpu_sc as plsc

N_TOKENS = 32768
DIM = 768
N_EXPERTS = 64
TOPK = 8
N_GROUPS = 8
GROUP_SIZE = N_EXPERTS // N_GROUPS
TOPK_GROUPS = 4

TILE = 4096

NEG_INF = float("-inf")


def _softmax_kernel(x_ref, wt_ref, p_ref, g_ref):
    t = x_ref.shape[0]
    scores = jnp.dot(x_ref[...], wt_ref[...], preferred_element_type=jnp.float32)
    s = scores.T
    smax = jnp.max(s, axis=0, keepdims=True)
    e = jnp.exp(s - smax)
    p = e / jnp.sum(e, axis=0, keepdims=True)
    p_ref[...] = p.T

    gs = jnp.max(p.reshape(N_GROUPS, GROUP_SIZE, t), axis=1)
    glane = lax.broadcasted_iota(jnp.int32, (N_GROUPS, t), 0)
    sels = []
    for k in range(TOPK_GROUPS):
        gm = jnp.max(gs, axis=0, keepdims=True)
        gsel = jnp.min(jnp.where(gs == gm, glane, N_GROUPS), axis=0,
                       keepdims=True)
        sels.append(gsel)
        if k != TOPK_GROUPS - 1:
            gs = jnp.where(glane == gsel, NEG_INF, gs)
    sels += [jnp.zeros_like(sels[0])] * (8 - TOPK_GROUPS)
    g_ref[...] = jnp.concatenate(sels, axis=0).T


def _route_body(p_hbm, g_hbm, w_hbm, i_hbm, p_v, g_v, w_v, i_v, *,
                rows_per, n_cores):
    cid = lax.axis_index("c")
    sid = lax.axis_index("s")
    wid = sid * n_cores + cid
    base = wid * rows_per

    pltpu.sync_copy(p_hbm.at[pl.ds(base * N_EXPERTS, rows_per * N_EXPERTS)], p_v)
    pltpu.sync_copy(g_hbm.at[pl.ds(base * 8, rows_per * 8)], g_v)

    lane = lax.iota(jnp.int32, 16)
    lane8 = lane & 7
    lane_hi = lane >> 3
    low8 = lane < 8

    @plsc.parallel_loop(0, rows_per, unroll=2)
    def _(r):
        gbase = jnp.full((16,), r * 8, jnp.int32)
        ga = plsc.load_gather(g_v, [gbase + lane_hi])
        gb = plsc.load_gather(g_v, [gbase + lane_hi + 2])
        cols_a = ga * GROUP_SIZE + lane8
        cols_b = gb * GROUP_SIZE + lane8
        rbase = jnp.full((16,), r * N_EXPERTS, jnp.int32)
        va = plsc.load_gather(p_v, [rbase + cols_a])
        vb = plsc.load_gather(p_v, [rbase + cols_b])
        ka, ia = plsc.sort_key_val(va, cols_a, descending=True)
        kb, ib = plsc.sort_key_val(vb, cols_b, descending=True)
        kc = jnp.where(low8, ka, kb.at[lane8].get(mode="promise_in_bounds"))
        ic = jnp.where(low8, ia, ib.at[lane8].get(mode="promise_in_bounds"))
        kf, if_ = plsc.sort_key_val(kc, ic, descending=True)
        out_idx = jnp.full((16,), r * TOPK, jnp.int32) + lane
        plsc.store_scatter(w_v, [out_idx], kf, mask=low8)
        plsc.store_scatter(i_v, [out_idx], if_, mask=low8)

    pltpu.sync_copy(w_v, w_hbm.at[pl.ds(base * TOPK, rows_per * TOPK)])
    pltpu.sync_copy(i_v, i_hbm.at[pl.ds(base * TOPK, rows_per * TOPK)])


@jax.jit
def kernel(x, weight):
    n = x.shape[0]
    wt = weight.T
    grid = (n // TILE,)
    p, g = pl.pallas_call(
        _softmax_kernel,
        grid=grid,
        in_specs=[
            pl.BlockSpec((TILE, DIM), lambda i: (i, 0)),
            pl.BlockSpec((DIM, N_EXPERTS), lambda i: (0, 0)),
        ],
        out_specs=[
            pl.BlockSpec((TILE, N_EXPERTS), lambda i: (i, 0)),
            pl.BlockSpec((TILE, 8), lambda i: (i, 0)),
        ],
        out_shape=[
            jax.ShapeDtypeStruct((n, N_EXPERTS), jnp.float32),
            jax.ShapeDtypeStruct((n, 8), jnp.int32),
        ],
    )(x, wt)

    info = plsc.get_sparse_core_info()
    nc, ns = info.num_cores, info.num_subcores
    rows_per = n // (nc * ns)
    mesh = plsc.VectorSubcoreMesh(core_axis_name="c", subcore_axis_name="s")
    w_flat, i_flat = pl.kernel(
        functools.partial(_route_body, rows_per=rows_per, n_cores=nc),
        out_type=[jax.ShapeDtypeStruct((n * TOPK,), jnp.float32),
                  jax.ShapeDtypeStruct((n * TOPK,), jnp.int32)],
        mesh=mesh,
        compiler_params=pltpu.CompilerParams(needs_layout_passes=False),
        scratch_types=[pltpu.VMEM((rows_per * N_EXPERTS,), jnp.float32),
                       pltpu.VMEM((rows_per * 8,), jnp.int32),
                       pltpu.VMEM((rows_per * TOPK,), jnp.float32),
                       pltpu.VMEM((rows_per * TOPK,), jnp.int32)],
    )(p.reshape(-1), g.reshape(-1))
    return w_flat.reshape(n, TOPK), i_flat.reshape(n, TOPK)


# speedup vs baseline: 1.4002x; 1.0050x over previous
import functools

import jax
import jax.numpy as jnp
from jax import lax
from jax.experimental import pallas as pl
from jax.experimental.pallas import tpu as pltpu
from jax.experimental.pallas import tpu_sc as plsc

N_TOKENS = 32768
DIM = 768
N_EXPERTS = 64
TOPK = 8
N_GROUPS = 8
GROUP_SIZE = N_EXPERTS // N_GROUPS
TOPK_GROUPS = 4

TILE = 4096

NEG_INF = float("-inf")


def _softmax_kernel(x_ref, wt_ref, p_ref, g_ref):
    t = x_ref.shape[0]
    scores = jnp.dot(x_ref[...], wt_ref[...], preferred_element_type=jnp.float32)
    s = scores.T
    smax = jnp.max(s, axis=0, keepdims=True)
    e = jnp.exp(s - smax)
    p = e / jnp.sum(e, axis=0, keepdims=True)
    p_ref[...] = p.T

    gs = jnp.max(p.reshape(N_GROUPS, GROUP_SIZE, t), axis=1)
    glane = lax.broadcasted_iota(jnp.int32, (N_GROUPS, t), 0)
    sels = []
    for k in range(TOPK_GROUPS):
        gm = jnp.max(gs, axis=0, keepdims=True)
        gsel = jnp.min(jnp.where(gs == gm, glane, N_GROUPS), axis=0,
                       keepdims=True)
        sels.append(gsel)
        if k != TOPK_GROUPS - 1:
            gs = jnp.where(glane == gsel, NEG_INF, gs)
    sels += [jnp.zeros_like(sels[0])] * (8 - TOPK_GROUPS)
    g_ref[...] = jnp.concatenate(sels, axis=0).T


def _route_body(p_hbm, g_hbm, w_hbm, i_hbm, p_v, g_v, w_v, i_v, *,
                rows_per, n_cores):
    cid = lax.axis_index("c")
    sid = lax.axis_index("s")
    wid = sid * n_cores + cid
    base = wid * rows_per

    pltpu.sync_copy(p_hbm.at[pl.ds(base * N_EXPERTS, rows_per * N_EXPERTS)], p_v)
    pltpu.sync_copy(g_hbm.at[pl.ds(base * 8, rows_per * 8)], g_v)

    lane = lax.iota(jnp.int32, 16)
    lane8 = lane & 7
    lane_hi = lane >> 3
    low8 = lane < 8

    @plsc.parallel_loop(0, rows_per, unroll=4)
    def _(r):
        gbase = jnp.full((16,), r * 8, jnp.int32)
        ga = plsc.load_gather(g_v, [gbase + lane_hi])
        gb = plsc.load_gather(g_v, [gbase + lane_hi + 2])
        cols_a = ga * GROUP_SIZE + lane8
        cols_b = gb * GROUP_SIZE + lane8
        rbase = jnp.full((16,), r * N_EXPERTS, jnp.int32)
        va = plsc.load_gather(p_v, [rbase + cols_a])
        vb = plsc.load_gather(p_v, [rbase + cols_b])
        ka, ia = plsc.sort_key_val(va, cols_a, descending=True)
        kb, ib = plsc.sort_key_val(vb, cols_b, descending=True)
        kc = jnp.where(low8, ka, kb.at[lane8].get(mode="promise_in_bounds"))
        ic = jnp.where(low8, ia, ib.at[lane8].get(mode="promise_in_bounds"))
        kf, if_ = plsc.sort_key_val(kc, ic, descending=True)
        out_idx = jnp.full((16,), r * TOPK, jnp.int32) + lane
        plsc.store_scatter(w_v, [out_idx], kf, mask=low8)
        plsc.store_scatter(i_v, [out_idx], if_, mask=low8)

    pltpu.sync_copy(w_v, w_hbm.at[pl.ds(base * TOPK, rows_per * TOPK)])
    pltpu.sync_copy(i_v, i_hbm.at[pl.ds(base * TOPK, rows_per * TOPK)])


@jax.jit
def kernel(x, weight):
    n = x.shape[0]
    wt = weight.T
    grid = (n // TILE,)
    p, g = pl.pallas_call(
        _softmax_kernel,
        grid=grid,
        in_specs=[
            pl.BlockSpec((TILE, DIM), lambda i: (i, 0)),
            pl.BlockSpec((DIM, N_EXPERTS), lambda i: (0, 0)),
        ],
        out_specs=[
            pl.BlockSpec((TILE, N_EXPERTS), lambda i: (i, 0)),
            pl.BlockSpec((TILE, 8), lambda i: (i, 0)),
        ],
        out_shape=[
            jax.ShapeDtypeStruct((n, N_EXPERTS), jnp.float32),
            jax.ShapeDtypeStruct((n, 8), jnp.int32),
        ],
    )(x, wt)

    info = plsc.get_sparse_core_info()
    nc, ns = info.num_cores, info.num_subcores
    rows_per = n // (nc * ns)
    mesh = plsc.VectorSubcoreMesh(core_axis_name="c", subcore_axis_name="s")
    w_flat, i_flat = pl.kernel(
        functools.partial(_route_body, rows_per=rows_per, n_cores=nc),
        out_type=[jax.ShapeDtypeStruct((n * TOPK,), jnp.float32),
                  jax.ShapeDtypeStruct((n * TOPK,), jnp.int32)],
        mesh=mesh,
        compiler_params=pltpu.CompilerParams(needs_layout_passes=False),
        scratch_types=[pltpu.VMEM((rows_per * N_EXPERTS,), jnp.float32),
                       pltpu.VMEM((rows_per * 8,), jnp.int32),
                       pltpu.VMEM((rows_per * TOPK,), jnp.float32),
                       pltpu.VMEM((rows_per * TOPK,), jnp.int32)],
    )(p.reshape(-1), g.reshape(-1))
    return w_flat.reshape(n, TOPK), i_flat.reshape(n, TOPK)
